# baseline (speedup 1.0000x reference)
import functools

import jax
import jax.numpy as jnp
from jax import lax
from jax.experimental import pallas as pl
from jax.experimental.pallas import tpu as pltpu
from jax.experimental.pallas import tpu_sc as plsc

_N = 10000
_E = 160000
_D = 256

_NC = 2
_NS = 16
_R = 320
_ACC_R = 328
_SEC = 1600
_NPAIR = _E // _SEC // 2
_RING = 4096
_CH = 32
_WIN = 512
_DRAIN_T = 896

_PREC = lax.Precision.HIGHEST


def _sc_agg_body(with_cnt, *refs):
    if with_cnt:
        (xp_hbm, src_hbm, dst_hbm, zacc_hbm, zcnt_hbm,
         sum_hbm, cnt_hbm,
         sec0, sec1, ring, eidx, srcv, rows0, rows1, rows2, rows3,
         acc, cntb,
         sS0, sS1, sE, sR0, sR1, sR2, sR3) = refs
    else:
        (xp_hbm, src_hbm, dst_hbm, zacc_hbm,
         sum_hbm,
         sec0, sec1, ring, eidx, srcv, rows0, rows1, rows2, rows3,
         acc,
         sS0, sS1, sE, sR0, sR1, sR2, sR3) = refs

    c = lax.axis_index("c")
    s = lax.axis_index("s")
    w = c * _NS + s
    lo = w * _R
    iot16 = lax.iota(jnp.int32, 16)
    lo_v = jnp.broadcast_to(lo, (16,))
    hi_v = jnp.broadcast_to(lo + _R, (16,))
    ones16 = jnp.full((16,), 1.0, jnp.float32)
    scrap_pk = jnp.broadcast_to(jnp.int32(_R << 18), (16,))

    pltpu.sync_copy(zacc_hbm, acc)
    if with_cnt:
        pltpu.sync_copy(zcnt_hbm, cntb)

    def scan(sec, e_base, off):
        @plsc.parallel_loop(0, _SEC // 16, carry=off, unroll=4)
        def body(i, off):
            d_v = sec[pl.ds(i * 16, 16)]
            mask = (d_v >= lo_v) & (d_v < hi_v)
            mi = mask.astype(jnp.int32)
            pos = (off + plsc.cumsum(mi) - 1) & (_RING - 1)
            e_v = e_base + i * 16 + iot16
            plsc.store_scatter(ring, [pos], ((d_v - lo_v) << 18) | e_v,
                               mask=mask)
            return off + plsc.all_reduce_population_count(mask)
        return body

    def accum_chunk(rstart, buf):
        for jj in range(2):
            pk_v = ring[pl.ds((rstart + jj * 16) & (_RING - 1), 16)]
            dloc_v = pk_v >> 18
            for j in range(16):
                dj = dloc_v[j]
                o = dj * _D
                if with_cnt:
                    plsc.addupdate(cntb.at[pl.ds(dj * 16, 16)], ones16)
                for k in range(_D // 16):
                    v = buf[jj * 16 + j, pl.ds(k * 16, 16)]
                    plsc.addupdate(acc.at[pl.ds(o + k * 16, 16)], v)

    def drain(ncons, nch):
        nwin = (nch + (_WIN // _CH) - 1) >> 4

        def win_body(wi, _):
            wstart = ncons + wi * _WIN
            wch = jnp.minimum(nch - wi * (_WIN // _CH), _WIN // _CH)

            def unp(u, _):
                v = ring[pl.ds((wstart + u * 16) & (_RING - 1), 16)]
                eidx[pl.ds(u * 16, 16)] = v & 0x3FFFF
                return 0
            lax.fori_loop(0, wch * 2, unp, 0)

            def zpad(u, _):
                eidx[pl.ds(u * 16, 16)] = jnp.zeros((16,), jnp.int32)
                return 0
            lax.fori_loop(wch * 2, _WIN // 16, zpad, 0)

            for g in range(_WIN // 128):
                @pl.when(g * 4 < wch)
                def _():
                    pltpu.async_copy(
                        src_hbm.at[eidx.at[pl.ds(g * 128, 128)]],
                        srcv.at[pl.ds(g * 128, 128)], sE)
            for g in range(_WIN // 128):
                @pl.when(g * 4 < wch)
                def _():
                    pltpu.make_async_copy(
                        src_hbm.at[eidx.at[pl.ds(g * 128, 128)]],
                        srcv.at[pl.ds(g * 128, 128)], sE).wait()

            rbs = (rows0, rows1, rows2, rows3)
            sems = (sR0, sR1, sR2, sR3)

            def rgather(j, buf, sem):
                return pltpu.async_copy(
                    xp_hbm.at[srcv.at[pl.ds(j * _CH, _CH)]], buf, sem)

            for q in range(4):
                rgather(q, rbs[q], sems[q])

            def quad_body(m, _):
                for q in range(4):
                    j = 4 * m + q
                    pltpu.make_async_copy(
                        xp_hbm.at[srcv.at[pl.ds(j * _CH, _CH)]],
                        rbs[q], sems[q]).wait()
                    accum_chunk(wstart + j * _CH, rbs[q])

                    @pl.when(j + 4 < wch)
                    def _():
                        rgather(j + 4, rbs[q], sems[q])
                return 0

            lax.fori_loop(0, wch >> 2, quad_body, 0)
            return 0

        lax.fori_loop(0, nwin, win_body, 0)

    pltpu.async_copy(dst_hbm.at[pl.ds(0, _SEC)], sec0, sS0)

    def sec_pair(k, carry):
        off, ncons = carry
        t0 = 2 * k

        def do_scan(offv):
            pltpu.make_async_copy(dst_hbm.at[pl.ds(t0 * _SEC, _SEC)],
                                  sec0, sS0).wait()
            cp = pltpu.async_copy(dst_hbm.at[pl.ds((t0 + 1) * _SEC, _SEC)],
                                  sec1, sS1)
            offv = scan(sec0, t0 * _SEC, offv)
            cp.wait()

            @pl.when(k + 1 < _NPAIR)
            def _():
                pltpu.async_copy(dst_hbm.at[pl.ds((t0 + 2) * _SEC, _SEC)],
                                 sec0, sS0)

            return scan(sec1, (t0 + 1) * _SEC, offv)

        off = lax.cond(k < _NPAIR, do_scan, lambda o: o, off)

        @pl.when(k == _NPAIR)
        def _():
            nprod = jnp.max(off)
            for i in range(9):
                pos = (nprod + i * 16 + iot16) & (_RING - 1)
                plsc.store_scatter(ring, [pos], scrap_pk)

        navail = jnp.max(off) - ncons
        nch = jnp.where(
            k == _NPAIR,
            (((jnp.max(off) + 127) & ~127) - ncons) >> 5,
            jnp.where(navail >= _DRAIN_T, (navail >> 7) << 2, 0))
        drain(ncons, nch)
        return off, ncons + nch * _CH

    lax.fori_loop(0, _NPAIR + 1, sec_pair,
                  (jnp.zeros((16,), jnp.int32), jnp.int32(0)))

    @pl.when(w < 31)
    def _():
        pltpu.sync_copy(acc.at[pl.ds(0, _R * _D)],
                        sum_hbm.at[pl.ds(lo * _D, _R * _D)])
        if with_cnt:
            pltpu.sync_copy(cntb.at[pl.ds(0, _R * 16)],
                            cnt_hbm.at[pl.ds(lo * 16, _R * 16)])

    @pl.when(w == 31)
    def _():
        nlast = _N - 31 * _R
        pltpu.sync_copy(acc.at[pl.ds(0, nlast * _D)],
                        sum_hbm.at[pl.ds(31 * _R * _D, nlast * _D)])
        if with_cnt:
            pltpu.sync_copy(cntb.at[pl.ds(0, nlast * 16)],
                            cnt_hbm.at[pl.ds(31 * _R * 16, nlast * 16)])


def _make_sc_agg(with_cnt):
    mesh = plsc.VectorSubcoreMesh(core_axis_name="c", subcore_axis_name="s",
                                  num_cores=_NC, num_subcores=_NS)
    if with_cnt:
        out_type = (jax.ShapeDtypeStruct((_N * _D,), jnp.float32),
                    jax.ShapeDtypeStruct((_N * 16,), jnp.float32))
    else:
        out_type = jax.ShapeDtypeStruct((_N * _D,), jnp.float32)
    scratch = [
        pltpu.VMEM((_SEC,), jnp.int32),
        pltpu.VMEM((_SEC,), jnp.int32),
        pltpu.VMEM((_RING,), jnp.int32),
        pltpu.VMEM((_WIN,), jnp.int32),
        pltpu.VMEM((_WIN,), jnp.int32),
        pltpu.VMEM((_CH, _D), jnp.float32),
        pltpu.VMEM((_CH, _D), jnp.float32),
        pltpu.VMEM((_CH, _D), jnp.float32),
        pltpu.VMEM((_CH, _D), jnp.float32),
        pltpu.VMEM((_ACC_R * _D,), jnp.float32),
    ]
    if with_cnt:
        scratch.append(pltpu.VMEM((_ACC_R * 16,), jnp.float32))
    scratch += [pltpu.SemaphoreType.DMA] * 7
    return pl.kernel(functools.partial(_sc_agg_body, with_cnt),
                     out_type=out_type, mesh=mesh, scratch_types=scratch,
                     compiler_params=pltpu.CompilerParams(
                         needs_layout_passes=False))


_make_sc_agg = functools.cache(_make_sc_agg)


_BLK = 2000


def _full(shape):
    return pl.BlockSpec(shape, lambda i: (0,) * len(shape))


def _rows(cols):
    return pl.BlockSpec((_BLK, cols), lambda i: (i, 0))


def _proj_body(x_ref, w_ref, b_ref, o_ref):
    o_ref[...] = jnp.maximum(
        jnp.dot(x_ref[...], w_ref[...], precision=_PREC,
                preferred_element_type=jnp.float32) + b_ref[...], 0.0)


def _tc_proj(x, w, b):
    return pl.pallas_call(
        _proj_body,
        grid=(_N // _BLK,),
        in_specs=[_rows(_D), _full((_D, _D)), _full((1, _D))],
        out_specs=_rows(_D),
        out_shape=jax.ShapeDtypeStruct((_N, _D), jnp.float32),
    )(x, w, b.reshape(1, _D))


def _mid_body(sum_ref, cnt_ref, x_ref, wll_ref, bll_ref, wlr_ref,
              g_ref, be_ref, w1_ref, b1_ref, h_ref, xp1_ref):
    inv = 1.0 / jnp.maximum(cnt_ref[:, 0:1], 1.0)
    mean = sum_ref[...] * inv
    out0 = (jnp.dot(mean, wll_ref[...], precision=_PREC,
                    preferred_element_type=jnp.float32) + bll_ref[...]
            + jnp.dot(x_ref[...], wlr_ref[...], precision=_PREC,
                      preferred_element_type=jnp.float32))
    a = jnp.maximum(out0, 0.0)
    mu = jnp.mean(a, axis=1, keepdims=True)
    d = a - mu
    var = jnp.mean(d * d, axis=1, keepdims=True)
    h = d * lax.rsqrt(var + 1e-5) * g_ref[...] + be_ref[...]
    h_ref[...] = h
    xp1_ref[...] = jnp.maximum(
        jnp.dot(h, w1_ref[...], precision=_PREC,
                preferred_element_type=jnp.float32) + b1_ref[...], 0.0)


def _tc_mid(summed, cnt, x, wll, bll, wlr, gamma, beta, w1, b1):
    return pl.pallas_call(
        _mid_body,
        grid=(_N // _BLK,),
        in_specs=[_rows(_D), _rows(16), _rows(_D),
                  _full((_D, _D)), _full((1, _D)), _full((_D, _D)),
                  _full((1, _D)), _full((1, _D)),
                  _full((_D, _D)), _full((1, _D))],
        out_specs=(_rows(_D), _rows(_D)),
        out_shape=(jax.ShapeDtypeStruct((_N, _D), jnp.float32),
                   jax.ShapeDtypeStruct((_N, _D), jnp.float32)),
    )(summed, cnt, x, wll, bll.reshape(1, _D), wlr,
      gamma.reshape(1, _D), beta.reshape(1, _D), w1, b1.reshape(1, _D))


def _final_body(sum_ref, cnt_ref, h_ref, wll_ref, bll_ref, wlr_ref, o_ref):
    inv = 1.0 / jnp.maximum(cnt_ref[:, 0:1], 1.0)
    mean = sum_ref[...] * inv
    o_ref[...] = (jnp.dot(mean, wll_ref[...], precision=_PREC,
                          preferred_element_type=jnp.float32) + bll_ref[...]
                  + jnp.dot(h_ref[...], wlr_ref[...], precision=_PREC,
                            preferred_element_type=jnp.float32))


def _tc_final(summed, cnt, h, wll, bll, wlr):
    return pl.pallas_call(
        _final_body,
        grid=(_N // _BLK,),
        in_specs=[_rows(_D), _rows(16), _rows(_D),
                  _full((_D, _D)), _full((1, _D)), _full((_D, _D))],
        out_specs=_rows(_D),
        out_shape=jax.ShapeDtypeStruct((_N, _D), jnp.float32),
    )(summed, cnt, h, wll, bll.reshape(1, _D), wlr)


def kernel(x, edge_index, l0_lin_W, l0_lin_b, l0_ll_W, l0_ll_b, l0_lr_W,
           ln_gamma, ln_beta, l1_lin_W, l1_lin_b, l1_ll_W, l1_ll_b, l1_lr_W):
    src = edge_index[0]
    dst = edge_index[1]
    zacc = jnp.zeros((_ACC_R * _D,), jnp.float32)
    zcnt = jnp.zeros((_ACC_R * 16,), jnp.float32)

    xp0 = _tc_proj(x, l0_lin_W, l0_lin_b)
    s0, cnt = _make_sc_agg(True)(xp0, src, dst, zacc, zcnt)
    summed0 = s0.reshape(_N, _D)
    cnt = cnt.reshape(_N, 16)
    h, xp1 = _tc_mid(summed0, cnt, x, l0_ll_W, l0_ll_b, l0_lr_W,
                     ln_gamma, ln_beta, l1_lin_W, l1_lin_b)
    s1 = _make_sc_agg(False)(xp1, src, dst, zacc)
    summed1 = s1.reshape(_N, _D)
    return _tc_final(summed1, cnt, h, l1_ll_W, l1_ll_b, l1_lr_W)


# speedup vs baseline: 1.0379x; 1.0379x over previous
import functools

import jax
import jax.numpy as jnp
from jax import lax
from jax.experimental import pallas as pl
from jax.experimental.pallas import tpu as pltpu
from jax.experimental.pallas import tpu_sc as plsc

_N = 10000
_E = 160000
_D = 256

_NC = 2
_NS = 16
_R = 320
_ACC_R = 321
_SEC = 1600
_NPAIR = _E // _SEC // 2
_RING = 4096
_CH = 32
_WIN = 1024
_DRAIN_T = 896

_PREC = lax.Precision.HIGHEST


def _sc_agg_body(with_cnt, *refs):
    if with_cnt:
        (xp_hbm, src_hbm, dst_hbm, zacc_hbm, zcnt_hbm,
         sum_hbm, cnt_hbm,
         sec0, sec1, ring, eidx, srcv, rows0, rows1, rows2, rows3,
         acc, cntb,
         sS0, sS1, sE, sR0, sR1, sR2, sR3) = refs
    else:
        (xp_hbm, src_hbm, dst_hbm, zacc_hbm,
         sum_hbm,
         sec0, sec1, ring, eidx, srcv, rows0, rows1, rows2, rows3,
         acc,
         sS0, sS1, sE, sR0, sR1, sR2, sR3) = refs

    c = lax.axis_index("c")
    s = lax.axis_index("s")
    w = c * _NS + s
    lo = w * _R
    iot16 = lax.iota(jnp.int32, 16)
    lo_v = jnp.broadcast_to(lo, (16,))
    hi_v = jnp.broadcast_to(lo + _R, (16,))
    ones16 = jnp.full((16,), 1.0, jnp.float32)
    scrap_pk = jnp.broadcast_to(jnp.int32(_R << 18), (16,))

    pltpu.sync_copy(zacc_hbm, acc)
    if with_cnt:
        pltpu.sync_copy(zcnt_hbm, cntb)

    def scan(sec, e_base, off):
        @plsc.parallel_loop(0, _SEC // 16, carry=off, unroll=4)
        def body(i, off):
            d_v = sec[pl.ds(i * 16, 16)]
            mask = (d_v >= lo_v) & (d_v < hi_v)
            mi = mask.astype(jnp.int32)
            pos = (off + plsc.cumsum(mi) - 1) & (_RING - 1)
            e_v = e_base + i * 16 + iot16
            plsc.store_scatter(ring, [pos], ((d_v - lo_v) << 18) | e_v,
                               mask=mask)
            return off + plsc.all_reduce_population_count(mask)
        return body

    def accum_chunk(rstart, buf):
        for jj in range(2):
            pk_v = ring[pl.ds((rstart + jj * 16) & (_RING - 1), 16)]
            dloc_v = pk_v >> 18
            for j in range(16):
                dj = dloc_v[j]
                o = dj * _D
                if with_cnt:
                    plsc.addupdate(cntb.at[pl.ds(dj * 16, 16)], ones16)
                for k in range(_D // 16):
                    v = buf[jj * 16 + j, pl.ds(k * 16, 16)]
                    plsc.addupdate(acc.at[pl.ds(o + k * 16, 16)], v)

    def drain(ncons, nch):
        nwin = (nch + (_WIN // _CH) - 1) >> 5

        def win_body(wi, _):
            wstart = ncons + wi * _WIN
            wch = jnp.minimum(nch - wi * (_WIN // _CH), _WIN // _CH)

            def unp(u, _):
                v = ring[pl.ds((wstart + u * 16) & (_RING - 1), 16)]
                eidx[pl.ds(u * 16, 16)] = v & 0x3FFFF
                return 0
            lax.fori_loop(0, wch * 2, unp, 0)

            def zpad(u, _):
                eidx[pl.ds(u * 16, 16)] = jnp.zeros((16,), jnp.int32)
                return 0
            lax.fori_loop(wch * 2, _WIN // 16, zpad, 0)

            for g in range(_WIN // 128):
                @pl.when(g * 4 < wch)
                def _():
                    pltpu.async_copy(
                        src_hbm.at[eidx.at[pl.ds(g * 128, 128)]],
                        srcv.at[pl.ds(g * 128, 128)], sE)
            for g in range(_WIN // 128):
                @pl.when(g * 4 < wch)
                def _():
                    pltpu.make_async_copy(
                        src_hbm.at[eidx.at[pl.ds(g * 128, 128)]],
                        srcv.at[pl.ds(g * 128, 128)], sE).wait()

            rbs = (rows0, rows1, rows2, rows3)
            sems = (sR0, sR1, sR2, sR3)

            def rgather(j, buf, sem):
                return pltpu.async_copy(
                    xp_hbm.at[srcv.at[pl.ds(j * _CH, _CH)]], buf, sem)

            for q in range(4):
                rgather(q, rbs[q], sems[q])

            def quad_body(m, _):
                for q in range(4):
                    j = 4 * m + q
                    pltpu.make_async_copy(
                        xp_hbm.at[srcv.at[pl.ds(j * _CH, _CH)]],
                        rbs[q], sems[q]).wait()
                    accum_chunk(wstart + j * _CH, rbs[q])

                    @pl.when(j + 4 < wch)
                    def _():
                        rgather(j + 4, rbs[q], sems[q])
                return 0

            lax.fori_loop(0, wch >> 2, quad_body, 0)
            return 0

        lax.fori_loop(0, nwin, win_body, 0)

    pltpu.async_copy(dst_hbm.at[pl.ds(0, _SEC)], sec0, sS0)

    def sec_pair(k, carry):
        off, ncons = carry
        t0 = 2 * k

        def do_scan(offv):
            pltpu.make_async_copy(dst_hbm.at[pl.ds(t0 * _SEC, _SEC)],
                                  sec0, sS0).wait()
            cp = pltpu.async_copy(dst_hbm.at[pl.ds((t0 + 1) * _SEC, _SEC)],
                                  sec1, sS1)
            offv = scan(sec0, t0 * _SEC, offv)
            cp.wait()

            @pl.when(k + 1 < _NPAIR)
            def _():
                pltpu.async_copy(dst_hbm.at[pl.ds((t0 + 2) * _SEC, _SEC)],
                                 sec0, sS0)

            return scan(sec1, (t0 + 1) * _SEC, offv)

        off = lax.cond(k < _NPAIR, do_scan, lambda o: o, off)

        @pl.when(k == _NPAIR)
        def _():
            nprod = jnp.max(off)
            for i in range(9):
                pos = (nprod + i * 16 + iot16) & (_RING - 1)
                plsc.store_scatter(ring, [pos], scrap_pk)

        navail = jnp.max(off) - ncons
        nch = jnp.where(
            k == _NPAIR,
            (((jnp.max(off) + 127) & ~127) - ncons) >> 5,
            jnp.where(navail >= _DRAIN_T, (navail >> 7) << 2, 0))
        drain(ncons, nch)
        return off, ncons + nch * _CH

    lax.fori_loop(0, _NPAIR + 1, sec_pair,
                  (jnp.zeros((16,), jnp.int32), jnp.int32(0)))

    @pl.when(w < 31)
    def _():
        pltpu.sync_copy(acc.at[pl.ds(0, _R * _D)],
                        sum_hbm.at[pl.ds(lo * _D, _R * _D)])
        if with_cnt:
            pltpu.sync_copy(cntb.at[pl.ds(0, _R * 16)],
                            cnt_hbm.at[pl.ds(lo * 16, _R * 16)])

    @pl.when(w == 31)
    def _():
        nlast = _N - 31 * _R
        pltpu.sync_copy(acc.at[pl.ds(0, nlast * _D)],
                        sum_hbm.at[pl.ds(31 * _R * _D, nlast * _D)])
        if with_cnt:
            pltpu.sync_copy(cntb.at[pl.ds(0, nlast * 16)],
                            cnt_hbm.at[pl.ds(31 * _R * 16, nlast * 16)])


def _make_sc_agg(with_cnt):
    mesh = plsc.VectorSubcoreMesh(core_axis_name="c", subcore_axis_name="s",
                                  num_cores=_NC, num_subcores=_NS)
    if with_cnt:
        out_type = (jax.ShapeDtypeStruct((_N * _D,), jnp.float32),
                    jax.ShapeDtypeStruct((_N * 16,), jnp.float32))
    else:
        out_type = jax.ShapeDtypeStruct((_N * _D,), jnp.float32)
    scratch = [
        pltpu.VMEM((_SEC,), jnp.int32),
        pltpu.VMEM((_SEC,), jnp.int32),
        pltpu.VMEM((_RING,), jnp.int32),
        pltpu.VMEM((_WIN,), jnp.int32),
        pltpu.VMEM((_WIN,), jnp.int32),
        pltpu.VMEM((_CH, _D), jnp.float32),
        pltpu.VMEM((_CH, _D), jnp.float32),
        pltpu.VMEM((_CH, _D), jnp.float32),
        pltpu.VMEM((_CH, _D), jnp.float32),
        pltpu.VMEM((_ACC_R * _D,), jnp.float32),
    ]
    if with_cnt:
        scratch.append(pltpu.VMEM((_ACC_R * 16,), jnp.float32))
    scratch += [pltpu.SemaphoreType.DMA] * 7
    return pl.kernel(functools.partial(_sc_agg_body, with_cnt),
                     out_type=out_type, mesh=mesh, scratch_types=scratch,
                     compiler_params=pltpu.CompilerParams(
                         needs_layout_passes=False))


_make_sc_agg = functools.cache(_make_sc_agg)


_BLK = 2000


def _full(shape):
    return pl.BlockSpec(shape, lambda i: (0,) * len(shape))


def _rows(cols):
    return pl.BlockSpec((_BLK, cols), lambda i: (i, 0))


def _proj_body(x_ref, w_ref, b_ref, o_ref):
    o_ref[...] = jnp.maximum(
        jnp.dot(x_ref[...], w_ref[...], precision=_PREC,
                preferred_element_type=jnp.float32) + b_ref[...], 0.0)


def _tc_proj(x, w, b):
    return pl.pallas_call(
        _proj_body,
        grid=(_N // _BLK,),
        in_specs=[_rows(_D), _full((_D, _D)), _full((1, _D))],
        out_specs=_rows(_D),
        out_shape=jax.ShapeDtypeStruct((_N, _D), jnp.float32),
    )(x, w, b.reshape(1, _D))


def _mid_body(sum_ref, cnt_ref, x_ref, wll_ref, bll_ref, wlr_ref,
              g_ref, be_ref, w1_ref, b1_ref, h_ref, xp1_ref):
    inv = 1.0 / jnp.maximum(cnt_ref[:, 0:1], 1.0)
    mean = sum_ref[...] * inv
    out0 = (jnp.dot(mean, wll_ref[...], precision=_PREC,
                    preferred_element_type=jnp.float32) + bll_ref[...]
            + jnp.dot(x_ref[...], wlr_ref[...], precision=_PREC,
                      preferred_element_type=jnp.float32))
    a = jnp.maximum(out0, 0.0)
    mu = jnp.mean(a, axis=1, keepdims=True)
    d = a - mu
    var = jnp.mean(d * d, axis=1, keepdims=True)
    h = d * lax.rsqrt(var + 1e-5) * g_ref[...] + be_ref[...]
    h_ref[...] = h
    xp1_ref[...] = jnp.maximum(
        jnp.dot(h, w1_ref[...], precision=_PREC,
                preferred_element_type=jnp.float32) + b1_ref[...], 0.0)


def _tc_mid(summed, cnt, x, wll, bll, wlr, gamma, beta, w1, b1):
    return pl.pallas_call(
        _mid_body,
        grid=(_N // _BLK,),
        in_specs=[_rows(_D), _rows(16), _rows(_D),
                  _full((_D, _D)), _full((1, _D)), _full((_D, _D)),
                  _full((1, _D)), _full((1, _D)),
                  _full((_D, _D)), _full((1, _D))],
        out_specs=(_rows(_D), _rows(_D)),
        out_shape=(jax.ShapeDtypeStruct((_N, _D), jnp.float32),
                   jax.ShapeDtypeStruct((_N, _D), jnp.float32)),
    )(summed, cnt, x, wll, bll.reshape(1, _D), wlr,
      gamma.reshape(1, _D), beta.reshape(1, _D), w1, b1.reshape(1, _D))


def _final_body(sum_ref, cnt_ref, h_ref, wll_ref, bll_ref, wlr_ref, o_ref):
    inv = 1.0 / jnp.maximum(cnt_ref[:, 0:1], 1.0)
    mean = sum_ref[...] * inv
    o_ref[...] = (jnp.dot(mean, wll_ref[...], precision=_PREC,
                          preferred_element_type=jnp.float32) + bll_ref[...]
                  + jnp.dot(h_ref[...], wlr_ref[...], precision=_PREC,
                            preferred_element_type=jnp.float32))


def _tc_final(summed, cnt, h, wll, bll, wlr):
    return pl.pallas_call(
        _final_body,
        grid=(_N // _BLK,),
        in_specs=[_rows(_D), _rows(16), _rows(_D),
                  _full((_D, _D)), _full((1, _D)), _full((_D, _D))],
        out_specs=_rows(_D),
        out_shape=jax.ShapeDtypeStruct((_N, _D), jnp.float32),
    )(summed, cnt, h, wll, bll.reshape(1, _D), wlr)


def kernel(x, edge_index, l0_lin_W, l0_lin_b, l0_ll_W, l0_ll_b, l0_lr_W,
           ln_gamma, ln_beta, l1_lin_W, l1_lin_b, l1_ll_W, l1_ll_b, l1_lr_W):
    src = edge_index[0]
    dst = edge_index[1]
    zacc = jnp.zeros((_ACC_R * _D,), jnp.float32)
    zcnt = jnp.zeros((_ACC_R * 16,), jnp.float32)

    xp0 = _tc_proj(x, l0_lin_W, l0_lin_b)
    s0, cnt = _make_sc_agg(True)(xp0, src, dst, zacc, zcnt)
    summed0 = s0.reshape(_N, _D)
    cnt = cnt.reshape(_N, 16)
    h, xp1 = _tc_mid(summed0, cnt, x, l0_ll_W, l0_ll_b, l0_lr_W,
                     ln_gamma, ln_beta, l1_lin_W, l1_lin_b)
    s1 = _make_sc_agg(False)(xp1, src, dst, zacc)
    summed1 = s1.reshape(_N, _D)
    return _tc_final(summed1, cnt, h, l1_ll_W, l1_ll_b, l1_lr_W)


# speedup vs baseline: 1.1831x; 1.1399x over previous
import functools

import jax
import jax.numpy as jnp
from jax import lax
from jax.experimental import pallas as pl
from jax.experimental.pallas import tpu as pltpu
from jax.experimental.pallas import tpu_sc as plsc

_N = 10000
_E = 160000
_D = 256

_NC = 2
_NS = 16
_R = 320
_ACC_R = 328
_SEC = 1600
_NPAIR = _E // _SEC // 2
_RING = 4096
_CH = 32
_WIN = 1024
_DRAIN_T = 896

_PREC = lax.Precision.DEFAULT


def _sc_agg_body(with_cnt, *refs):
    if with_cnt:
        (xp_hbm, src_hbm, dst_hbm, zacc_hbm, zcnt_hbm,
         sum_hbm, cnt_hbm,
         sec0, sec1, ring, eidx, srcv, rows0, rows1, acc, cntb,
         sS0, sS1, sE, sR0, sR1) = refs
    else:
        (xp_hbm, src_hbm, dst_hbm, zacc_hbm,
         sum_hbm,
         sec0, sec1, ring, eidx, srcv, rows0, rows1, acc,
         sS0, sS1, sE, sR0, sR1) = refs

    c = lax.axis_index("c")
    s = lax.axis_index("s")
    w = c * _NS + s
    lo = w * _R
    iot16 = lax.iota(jnp.int32, 16)
    lo_v = jnp.broadcast_to(lo, (16,))
    hi_v = jnp.broadcast_to(lo + _R, (16,))
    ones16 = jnp.full((16,), 1.0, jnp.float32)
    scrap_pk = jnp.broadcast_to(jnp.int32(_R << 18), (16,))

    pltpu.sync_copy(zacc_hbm, acc)
    if with_cnt:
        pltpu.sync_copy(zcnt_hbm, cntb)

    def scan(sec, e_base, off):
        @plsc.parallel_loop(0, _SEC // 16, carry=off, unroll=4)
        def body(i, off):
            d_v = sec[pl.ds(i * 16, 16)]
            mask = (d_v >= lo_v) & (d_v < hi_v)
            mi = mask.astype(jnp.int32)
            pos = (off + plsc.cumsum(mi) - 1) & (_RING - 1)
            e_v = e_base + i * 16 + iot16
            plsc.store_scatter(ring, [pos], ((d_v - lo_v) << 18) | e_v,
                               mask=mask)
            return off + plsc.all_reduce_population_count(mask)
        return body

    def accum_chunk(rstart, buf):
        for jj in range(2):
            pk_v = ring[pl.ds((rstart + jj * 16) & (_RING - 1), 16)]
            dloc_v = pk_v >> 18
            for j in range(16):
                dj = dloc_v[j]
                o = dj * _D
                if with_cnt:
                    plsc.addupdate(cntb.at[pl.ds(dj * 16, 16)], ones16)
                for k in range(_D // 16):
                    v = buf[jj * 16 + j, pl.ds(k * 16, 16)]
                    plsc.addupdate(acc.at[pl.ds(o + k * 16, 16)], v)

    def drain(ncons, nch):
        nwin = (nch + (_WIN // _CH) - 1) >> 5

        def win_body(wi, _):
            wstart = ncons + wi * _WIN
            wch = jnp.minimum(nch - wi * (_WIN // _CH), _WIN // _CH)

            def unp(u, _):
                v = ring[pl.ds((wstart + u * 16) & (_RING - 1), 16)]
                eidx[pl.ds(u * 16, 16)] = v & 0x3FFFF
                return 0
            lax.fori_loop(0, wch * 2, unp, 0)

            def zpad(u, _):
                eidx[pl.ds(u * 16, 16)] = jnp.zeros((16,), jnp.int32)
                return 0
            lax.fori_loop(wch * 2, _WIN // 16, zpad, 0)

            for g in range(_WIN // 128):
                @pl.when(g * 4 < wch)
                def _():
                    pltpu.async_copy(
                        src_hbm.at[eidx.at[pl.ds(g * 128, 128)]],
                        srcv.at[pl.ds(g * 128, 128)], sE)
            for g in range(_WIN // 128):
                @pl.when(g * 4 < wch)
                def _():
                    pltpu.make_async_copy(
                        src_hbm.at[eidx.at[pl.ds(g * 128, 128)]],
                        srcv.at[pl.ds(g * 128, 128)], sE).wait()

            def rgather(j, buf, sem):
                return pltpu.async_copy(
                    xp_hbm.at[srcv.at[pl.ds(j * _CH, _CH)]], buf, sem)

            rgather(0, rows0, sR0)

            def pair_body(m, _):
                j0 = 2 * m
                j1 = j0 + 1
                pltpu.make_async_copy(
                    xp_hbm.at[srcv.at[pl.ds(j0 * _CH, _CH)]],
                    rows0, sR0).wait()
                cp1 = rgather(j1, rows1, sR1)
                accum_chunk(wstart + j0 * _CH, rows0)
                cp1.wait()

                @pl.when(j0 + 2 < wch)
                def _():
                    rgather(j0 + 2, rows0, sR0)

                accum_chunk(wstart + j1 * _CH, rows1)
                return 0

            lax.fori_loop(0, wch >> 1, pair_body, 0)
            return 0

        lax.fori_loop(0, nwin, win_body, 0)

    def maybe_drain(off, ncons):
        navail = jnp.max(off) - ncons
        nch = jnp.where(navail >= _DRAIN_T, (navail >> 6) << 1, 0)
        drain(ncons, nch)
        return ncons + nch * _CH

    pltpu.async_copy(dst_hbm.at[pl.ds(0, _SEC)], sec0, sS0)

    def sec_pair(k, carry):
        off, ncons = carry
        t0 = 2 * k
        pltpu.make_async_copy(dst_hbm.at[pl.ds(t0 * _SEC, _SEC)],
                              sec0, sS0).wait()
        cp = pltpu.async_copy(dst_hbm.at[pl.ds((t0 + 1) * _SEC, _SEC)],
                              sec1, sS1)
        off = scan(sec0, t0 * _SEC, off)
        cp.wait()

        @pl.when(k + 1 < _NPAIR)
        def _():
            pltpu.async_copy(dst_hbm.at[pl.ds((t0 + 2) * _SEC, _SEC)],
                             sec0, sS0)

        off = scan(sec1, (t0 + 1) * _SEC, off)
        ncons = maybe_drain(off, ncons)
        return off, ncons

    off, ncons = lax.fori_loop(0, _NPAIR, sec_pair,
                               (jnp.zeros((16,), jnp.int32), jnp.int32(0)))

    nprod = jnp.max(off)
    for i in range(5):
        pos = (nprod + i * 16 + iot16) & (_RING - 1)
        plsc.store_scatter(ring, [pos], scrap_pk)
    nprod_r = (nprod + 63) & ~63
    drain(ncons, (nprod_r - ncons) >> 5)

    @pl.when(w < 31)
    def _():
        pltpu.sync_copy(acc.at[pl.ds(0, _R * _D)],
                        sum_hbm.at[pl.ds(lo * _D, _R * _D)])
        if with_cnt:
            pltpu.sync_copy(cntb.at[pl.ds(0, _R * 16)],
                            cnt_hbm.at[pl.ds(lo * 16, _R * 16)])

    @pl.when(w == 31)
    def _():
        nlast = _N - 31 * _R
        pltpu.sync_copy(acc.at[pl.ds(0, nlast * _D)],
                        sum_hbm.at[pl.ds(31 * _R * _D, nlast * _D)])
        if with_cnt:
            pltpu.sync_copy(cntb.at[pl.ds(0, nlast * 16)],
                            cnt_hbm.at[pl.ds(31 * _R * 16, nlast * 16)])


def _make_sc_agg(with_cnt):
    mesh = plsc.VectorSubcoreMesh(core_axis_name="c", subcore_axis_name="s",
                                  num_cores=_NC, num_subcores=_NS)
    if with_cnt:
        out_type = (jax.ShapeDtypeStruct((_N * _D,), jnp.float32),
                    jax.ShapeDtypeStruct((_N * 16,), jnp.float32))
    else:
        out_type = jax.ShapeDtypeStruct((_N * _D,), jnp.float32)
    scratch = [
        pltpu.VMEM((_SEC,), jnp.int32),
        pltpu.VMEM((_SEC,), jnp.int32),
        pltpu.VMEM((_RING,), jnp.int32),
        pltpu.VMEM((_WIN,), jnp.int32),
        pltpu.VMEM((_WIN,), jnp.int32),
        pltpu.VMEM((_CH, _D), jnp.float32),
        pltpu.VMEM((_CH, _D), jnp.float32),
        pltpu.VMEM((_ACC_R * _D,), jnp.float32),
    ]
    if with_cnt:
        scratch.append(pltpu.VMEM((_ACC_R * 16,), jnp.float32))
    scratch += [pltpu.SemaphoreType.DMA] * 5
    return pl.kernel(functools.partial(_sc_agg_body, with_cnt),
                     out_type=out_type, mesh=mesh, scratch_types=scratch,
                     compiler_params=pltpu.CompilerParams(
                         needs_layout_passes=False))


_make_sc_agg = functools.cache(_make_sc_agg)


_BLK = 2000


def _full(shape):
    return pl.BlockSpec(shape, lambda i: (0,) * len(shape))


def _rows(cols):
    return pl.BlockSpec((_BLK, cols), lambda i: (i, 0))


def _proj_body(x_ref, w_ref, b_ref, o_ref):
    o_ref[...] = jnp.maximum(
        jnp.dot(x_ref[...], w_ref[...], precision=_PREC,
                preferred_element_type=jnp.float32) + b_ref[...], 0.0)


def _tc_proj(x, w, b):
    return pl.pallas_call(
        _proj_body,
        grid=(_N // _BLK,),
        in_specs=[_rows(_D), _full((_D, _D)), _full((1, _D))],
        out_specs=_rows(_D),
        out_shape=jax.ShapeDtypeStruct((_N, _D), jnp.float32),
    )(x, w, b.reshape(1, _D))


def _mid_body(sum_ref, cnt_ref, x_ref, wll_ref, bll_ref, wlr_ref,
              g_ref, be_ref, w1_ref, b1_ref, h_ref, xp1_ref):
    inv = 1.0 / jnp.maximum(cnt_ref[:, 0:1], 1.0)
    mean = sum_ref[...] * inv
    out0 = (jnp.dot(mean, wll_ref[...], precision=_PREC,
                    preferred_element_type=jnp.float32) + bll_ref[...]
            + jnp.dot(x_ref[...], wlr_ref[...], precision=_PREC,
                      preferred_element_type=jnp.float32))
    a = jnp.maximum(out0, 0.0)
    mu = jnp.mean(a, axis=1, keepdims=True)
    d = a - mu
    var = jnp.mean(d * d, axis=1, keepdims=True)
    h = d * lax.rsqrt(var + 1e-5) * g_ref[...] + be_ref[...]
    h_ref[...] = h
    xp1_ref[...] = jnp.maximum(
        jnp.dot(h, w1_ref[...], precision=_PREC,
                preferred_element_type=jnp.float32) + b1_ref[...], 0.0)


def _tc_mid(summed, cnt, x, wll, bll, wlr, gamma, beta, w1, b1):
    return pl.pallas_call(
        _mid_body,
        grid=(_N // _BLK,),
        in_specs=[_rows(_D), _rows(16), _rows(_D),
                  _full((_D, _D)), _full((1, _D)), _full((_D, _D)),
                  _full((1, _D)), _full((1, _D)),
                  _full((_D, _D)), _full((1, _D))],
        out_specs=(_rows(_D), _rows(_D)),
        out_shape=(jax.ShapeDtypeStruct((_N, _D), jnp.float32),
                   jax.ShapeDtypeStruct((_N, _D), jnp.float32)),
    )(summed, cnt, x, wll, bll.reshape(1, _D), wlr,
      gamma.reshape(1, _D), beta.reshape(1, _D), w1, b1.reshape(1, _D))


def _final_body(sum_ref, cnt_ref, h_ref, wll_ref, bll_ref, wlr_ref, o_ref):
    inv = 1.0 / jnp.maximum(cnt_ref[:, 0:1], 1.0)
    mean = sum_ref[...] * inv
    o_ref[...] = (jnp.dot(mean, wll_ref[...], precision=_PREC,
                          preferred_element_type=jnp.float32) + bll_ref[...]
                  + jnp.dot(h_ref[...], wlr_ref[...], precision=_PREC,
                            preferred_element_type=jnp.float32))


def _tc_final(summed, cnt, h, wll, bll, wlr):
    return pl.pallas_call(
        _final_body,
        grid=(_N // _BLK,),
        in_specs=[_rows(_D), _rows(16), _rows(_D),
                  _full((_D, _D)), _full((1, _D)), _full((_D, _D))],
        out_specs=_rows(_D),
        out_shape=jax.ShapeDtypeStruct((_N, _D), jnp.float32),
    )(summed, cnt, h, wll, bll.reshape(1, _D), wlr)


def kernel(x, edge_index, l0_lin_W, l0_lin_b, l0_ll_W, l0_ll_b, l0_lr_W,
           ln_gamma, ln_beta, l1_lin_W, l1_lin_b, l1_ll_W, l1_ll_b, l1_lr_W):
    src = edge_index[0]
    dst = edge_index[1]
    zacc = jnp.zeros((_ACC_R * _D,), jnp.float32)
    zcnt = jnp.zeros((_ACC_R * 16,), jnp.float32)

    xp0 = _tc_proj(x, l0_lin_W, l0_lin_b)
    s0, cnt = _make_sc_agg(True)(xp0, src, dst, zacc, zcnt)
    summed0 = s0.reshape(_N, _D)
    cnt = cnt.reshape(_N, 16)
    h, xp1 = _tc_mid(summed0, cnt, x, l0_ll_W, l0_ll_b, l0_lr_W,
                     ln_gamma, ln_beta, l1_lin_W, l1_lin_b)
    s1 = _make_sc_agg(False)(xp1, src, dst, zacc)
    summed1 = s1.reshape(_N, _D)
    return _tc_final(summed1, cnt, h, l1_ll_W, l1_ll_b, l1_lr_W)
